# baseline (speedup 1.0000x reference)
import functools

import jax
import jax.numpy as jnp
from jax import lax
from jax.experimental import pallas as pl
from jax.experimental.pallas import tpu as pltpu
from jax.experimental.pallas import tpu_sc as plsc

S, D, E, K, H, HD, HID = 2048, 768, 8, 2, 12, 64, 3072
GT = 512
BT = 256
HC = 2
HIDC = HID // HC
NPAIR = S * K
NB = NPAIR // BT + E
PMAX = NB * BT
NW = 32
PAIRS_PER = NPAIR // NW


def _gate_route_body(x_ref, emb_ref, wq_ref, bq_ref, wk_ref, bk_ref, wv_ref,
                     bv_ref, wo_ref, bo_ref, wg_ref, bg_ref, topw_ref,
                     pos0_ref, pos1_ref, blk_e_ref, blk_v_ref):
    xb = x_ref[...]
    q = jnp.dot(xb, wq_ref[...], preferred_element_type=jnp.float32) + bq_ref[...]
    emb = emb_ref[...]
    k = jnp.dot(emb, wk_ref[...], preferred_element_type=jnp.float32) + bk_ref[...]
    v = jnp.dot(emb, wv_ref[...], preferred_element_type=jnp.float32) + bv_ref[...]
    wog = jnp.dot(wo_ref[...], wg_ref[...], preferred_element_type=jnp.float32)
    bog = jnp.dot(bo_ref[...], wg_ref[...], preferred_element_type=jnp.float32) + bg_ref[...]

    logits = jnp.zeros((S, E), jnp.float32)
    inv = 1.0 / (HD ** 0.5)
    for h in range(H):
        sl = slice(h * HD, (h + 1) * HD)
        qh = q[:, sl]
        kh = k[:, sl]
        vh = v[:, sl]
        s = lax.dot_general(qh, kh, (((1,), (1,)), ((), ())),
                            preferred_element_type=jnp.float32) * inv
        s = s - jnp.max(s, axis=1, keepdims=True)
        p = jnp.exp(s)
        p = p / jnp.sum(p, axis=1, keepdims=True)
        ao = jnp.dot(p, vh, preferred_element_type=jnp.float32)
        logits = logits + jnp.dot(ao, wog[sl, :], preferred_element_type=jnp.float32)
    logits = logits + bog

    logits = logits - jnp.max(logits, axis=1, keepdims=True)
    ep = jnp.exp(logits)
    probs = ep / jnp.sum(ep, axis=1, keepdims=True)

    eio = lax.broadcasted_iota(jnp.int32, (S, E), 1)
    m1 = jnp.max(probs, axis=1, keepdims=True)
    i1 = jnp.min(jnp.where(probs == m1, eio, E), axis=1, keepdims=True)
    probs2 = jnp.where(eio == i1, -1.0, probs)
    m2 = jnp.max(probs2, axis=1, keepdims=True)
    i2 = jnp.min(jnp.where(probs2 == m2, eio, E), axis=1, keepdims=True)
    denom = m1 + m2 + 1e-9
    topw_ref[...] = jnp.concatenate([m1 / denom, m2 / denom], axis=1)

    oh = jnp.concatenate([(i1 == eio[:, :]).astype(jnp.int32),
                          (i2 == eio[:, :]).astype(jnp.int32)], axis=0)
    run = oh
    sh = 1
    while sh < NPAIR:
        z = jnp.zeros((sh, E), jnp.int32)
        run = run + jnp.concatenate([z, run[:NPAIR - sh, :]], axis=0)
        sh *= 2
    rank = run - oh
    counts = run[NPAIR - 1:NPAIR, :]
    padded = ((counts + (BT - 1)) // BT) * BT
    r8 = lax.broadcasted_iota(jnp.int32, (E, E), 0)
    c8 = lax.broadcasted_iota(jnp.int32, (E, E), 1)
    ltri = (r8 < c8).astype(jnp.float32)
    start = jnp.dot(padded.astype(jnp.float32), ltri,
                    preferred_element_type=jnp.float32).astype(jnp.int32)
    pos = jnp.sum(jnp.where(oh > 0, rank + start, 0), axis=1, keepdims=True)
    pos0_ref[...] = pos[:S, :]
    pos1_ref[...] = pos[S:, :]

    bcnt = padded // BT
    bend = (start // BT) + bcnt
    total = jnp.sum(bcnt, axis=1, keepdims=True)
    laste = jnp.max(jnp.where(bcnt > 0, lax.broadcasted_iota(jnp.int32, (1, E), 1), 0),
                    axis=1, keepdims=True)
    bio = lax.broadcasted_iota(jnp.int32, (NB, E), 0)
    cnt_before = jnp.sum((jnp.broadcast_to(bend, (NB, E)) <= bio).astype(jnp.int32),
                         axis=1, keepdims=True)
    bvalid = (lax.broadcasted_iota(jnp.int32, (NB, 1), 0) < total).astype(jnp.int32)
    blk_e_ref[...] = jnp.where(bvalid > 0, cnt_before, laste)
    blk_v_ref[...] = bvalid


def _gate_route(x2, emb, Wq, bq, Wk, bk, Wv, bv, Wo, bo, Wg, bg):
    return pl.pallas_call(
        _gate_route_body,
        out_shape=[
            jax.ShapeDtypeStruct((S, K), jnp.float32),
            jax.ShapeDtypeStruct((S, 1), jnp.int32),
            jax.ShapeDtypeStruct((S, 1), jnp.int32),
            jax.ShapeDtypeStruct((NB, 1), jnp.int32),
            jax.ShapeDtypeStruct((NB, 1), jnp.int32),
        ],
    )(x2, emb, Wq, bq, Wk, bk, Wv, bv, Wo, bo, Wg, bg)


TOK_PER = S // NW


@functools.cache
def _sc_kernels():
    mesh = plsc.VectorSubcoreMesh(core_axis_name="c", subcore_axis_name="s")

    @functools.partial(
        pl.kernel,
        out_type=jax.ShapeDtypeStruct((PMAX, D), jnp.float32),
        mesh=mesh,
        scratch_types=[
            pltpu.VMEM((TOK_PER,), jnp.int32),
            pltpu.VMEM((TOK_PER,), jnp.int32),
            pltpu.VMEM((TOK_PER, D), jnp.float32),
            pltpu.SemaphoreType.DMA,
        ],
    )
    def _dispatch(x_hbm, pos0_hbm, pos1_hbm, xg_hbm, pos0_v, pos1_v, rows_v,
                  sem):
        wid = lax.axis_index("s") * 2 + lax.axis_index("c")
        base = wid * TOK_PER
        pltpu.sync_copy(pos0_hbm.at[pl.ds(base, TOK_PER)], pos0_v)
        pltpu.sync_copy(pos1_hbm.at[pl.ds(base, TOK_PER)], pos1_v)
        pltpu.sync_copy(x_hbm.at[pl.ds(base, TOK_PER)], rows_v)
        d0 = pltpu.async_copy(rows_v, xg_hbm.at[pos0_v], sem)
        d1 = pltpu.async_copy(rows_v, xg_hbm.at[pos1_v], sem)
        d0.wait()
        d1.wait()

    @functools.partial(
        pl.kernel,
        out_type=(
            jax.ShapeDtypeStruct((S, D), jnp.float32),
            jax.ShapeDtypeStruct((S, D), jnp.float32),
        ),
        mesh=mesh,
        scratch_types=[
            pltpu.VMEM((TOK_PER,), jnp.int32),
            pltpu.VMEM((TOK_PER,), jnp.int32),
            pltpu.VMEM((TOK_PER, D), jnp.float32),
            pltpu.VMEM((TOK_PER, D), jnp.float32),
            pltpu.SemaphoreType.DMA,
        ],
    )
    def _combine_gather(y_hbm, pos0_hbm, pos1_hbm, y0_hbm, y1_hbm, pos0_v,
                        pos1_v, rows0_v, rows1_v, sem):
        wid = lax.axis_index("s") * 2 + lax.axis_index("c")
        base = wid * TOK_PER
        pltpu.sync_copy(pos0_hbm.at[pl.ds(base, TOK_PER)], pos0_v)
        pltpu.sync_copy(pos1_hbm.at[pl.ds(base, TOK_PER)], pos1_v)
        g0 = pltpu.async_copy(y_hbm.at[pos0_v], rows0_v, sem)
        g1 = pltpu.async_copy(y_hbm.at[pos1_v], rows1_v, sem)
        g0.wait()
        g1.wait()
        pltpu.sync_copy(rows0_v, y0_hbm.at[pl.ds(base, TOK_PER)])
        pltpu.sync_copy(rows1_v, y1_hbm.at[pl.ds(base, TOK_PER)])

    return _dispatch, _combine_gather


def _moe_body(blk_e_ref, blk_v_ref, xg_ref, w1_ref, b1_ref, w2_ref, b2_ref,
              y_ref):
    i = pl.program_id(0)
    c = pl.program_id(1)

    @pl.when(blk_v_ref[i] > 0)
    def _():
        h = jnp.dot(xg_ref[...], w1_ref[0], preferred_element_type=jnp.float32)
        h = jnp.maximum(h + b1_ref[0], 0.0)
        y = jnp.dot(h, w2_ref[0], preferred_element_type=jnp.float32)

        @pl.when(c == 0)
        def _():
            y_ref[...] = y + b2_ref[0]

        @pl.when(c > 0)
        def _():
            y_ref[...] = y_ref[...] + y


def _moe(blk_e, blk_v, xg, W1, b1, W2, b2):
    grid_spec = pltpu.PrefetchScalarGridSpec(
        num_scalar_prefetch=2,
        grid=(NB, HC),
        in_specs=[
            pl.BlockSpec((BT, D), lambda i, c, be, bv: (i, 0)),
            pl.BlockSpec((1, D, HIDC), lambda i, c, be, bv: (be[i], 0, c)),
            pl.BlockSpec((1, 1, HIDC), lambda i, c, be, bv: (be[i], 0, c)),
            pl.BlockSpec((1, HIDC, D), lambda i, c, be, bv: (be[i], c, 0)),
            pl.BlockSpec((1, 1, D), lambda i, c, be, bv: (be[i], 0, 0)),
        ],
        out_specs=pl.BlockSpec((BT, D), lambda i, c, be, bv: (i, 0)),
    )
    return pl.pallas_call(
        _moe_body,
        grid_spec=grid_spec,
        out_shape=jax.ShapeDtypeStruct((PMAX, D), jnp.float32),
    )(blk_e, blk_v, xg, W1, b1, W2, b2)


def _comb_body(y0_ref, y1_ref, w_ref, o_ref):
    w = w_ref[...]
    o_ref[...] = y0_ref[...] * w[:, 0:1] + y1_ref[...] * w[:, 1:2]


def _comb(y0, y1, topw):
    return pl.pallas_call(
        _comb_body,
        grid=(S // GT,),
        in_specs=[
            pl.BlockSpec((GT, D), lambda i: (i, 0)),
            pl.BlockSpec((GT, D), lambda i: (i, 0)),
            pl.BlockSpec((GT, K), lambda i: (i, 0)),
        ],
        out_specs=pl.BlockSpec((GT, D), lambda i: (i, 0)),
        out_shape=jax.ShapeDtypeStruct((S, D), jnp.float32),
    )(y0, y1, topw)


def kernel(x, expert_emb, Wq, bq, Wk, bk, Wv, bv, Wo, bo, Wg, bg, W1, b1, W2,
           b2):
    x2 = x.reshape(S, D)
    topw, pos0c, pos1c, blk_e2, blk_v2 = _gate_route(
        x2, expert_emb, Wq, bq.reshape(1, D), Wk, bk.reshape(1, D), Wv,
        bv.reshape(1, D), Wo, bo.reshape(1, D), Wg, bg.reshape(1, E))
    pos0 = pos0c.reshape(S)
    pos1 = pos1c.reshape(S)
    dispatch_fn, combine_fn = _sc_kernels()
    xg = dispatch_fn(x2, pos0, pos1)
    y = _moe(blk_e2.reshape(NB), blk_v2.reshape(NB), xg, W1,
             b1.reshape(E, 1, HID), W2, b2.reshape(E, 1, D))
    y0, y1 = combine_fn(y, pos0, pos1)
    out = _comb(y0, y1, topw)
    return out.reshape(1, S, D)


# speedup vs baseline: 1.5866x; 1.5866x over previous
import functools

import jax
import jax.numpy as jnp
from jax import lax
from jax.experimental import pallas as pl
from jax.experimental.pallas import tpu as pltpu
from jax.experimental.pallas import tpu_sc as plsc

S, D, E, K, H, HD, HID = 2048, 768, 8, 2, 12, 64, 3072
GT = 512
BT = 256
HC = 2
HIDC = HID // HC
NPAIR = S * K
NB = NPAIR // BT + E
PMAX = NB * BT
NW = 32
PAIRS_PER = NPAIR // NW


def _gate_route_body(x_ref, emb_ref, wq_ref, bq_ref, wk_ref, bk_ref, wv_ref,
                     bv_ref, wo_ref, bo_ref, wg_ref, bg_ref, topw_ref,
                     pos0_ref, pos1_ref, blk_e_ref, blk_v_ref):
    xb = x_ref[...]
    q = jnp.dot(xb, wq_ref[...], preferred_element_type=jnp.float32) + bq_ref[...]
    emb = emb_ref[...]
    k = jnp.dot(emb, wk_ref[...], preferred_element_type=jnp.float32) + bk_ref[...]
    v = jnp.dot(emb, wv_ref[...], preferred_element_type=jnp.float32) + bv_ref[...]
    wog = jnp.dot(wo_ref[...], wg_ref[...], preferred_element_type=jnp.float32)
    bog = jnp.dot(bo_ref[...], wg_ref[...], preferred_element_type=jnp.float32) + bg_ref[...]

    logits = jnp.zeros((S, E), jnp.float32)
    inv = 1.0 / (HD ** 0.5)
    for h in range(H):
        sl = slice(h * HD, (h + 1) * HD)
        qh = q[:, sl]
        kh = k[:, sl]
        vh = v[:, sl]
        s = lax.dot_general(qh, kh, (((1,), (1,)), ((), ())),
                            preferred_element_type=jnp.float32) * inv
        s = s - jnp.max(s, axis=1, keepdims=True)
        p = jnp.exp(s)
        p = p / jnp.sum(p, axis=1, keepdims=True)
        ao = jnp.dot(p, vh, preferred_element_type=jnp.float32)
        logits = logits + jnp.dot(ao, wog[sl, :], preferred_element_type=jnp.float32)
    logits = logits + bog

    logits = logits - jnp.max(logits, axis=1, keepdims=True)
    ep = jnp.exp(logits)
    probs = ep / jnp.sum(ep, axis=1, keepdims=True)

    eio = lax.broadcasted_iota(jnp.int32, (S, E), 1)
    m1 = jnp.max(probs, axis=1, keepdims=True)
    i1 = jnp.min(jnp.where(probs == m1, eio, E), axis=1, keepdims=True)
    probs2 = jnp.where(eio == i1, -1.0, probs)
    m2 = jnp.max(probs2, axis=1, keepdims=True)
    i2 = jnp.min(jnp.where(probs2 == m2, eio, E), axis=1, keepdims=True)
    denom = m1 + m2 + 1e-9
    topw_ref[...] = jnp.concatenate([m1 / denom, m2 / denom], axis=1)

    oh = jnp.concatenate([(i1 == eio[:, :]).astype(jnp.int32),
                          (i2 == eio[:, :]).astype(jnp.int32)], axis=0)
    run = oh
    sh = 1
    while sh < NPAIR:
        z = jnp.zeros((sh, E), jnp.int32)
        run = run + jnp.concatenate([z, run[:NPAIR - sh, :]], axis=0)
        sh *= 2
    rank = run - oh
    counts = run[NPAIR - 1:NPAIR, :]
    padded = ((counts + (BT - 1)) // BT) * BT
    r8 = lax.broadcasted_iota(jnp.int32, (E, E), 0)
    c8 = lax.broadcasted_iota(jnp.int32, (E, E), 1)
    ltri = (r8 < c8).astype(jnp.float32)
    start = jnp.dot(padded.astype(jnp.float32), ltri,
                    preferred_element_type=jnp.float32).astype(jnp.int32)
    pos = jnp.sum(jnp.where(oh > 0, rank + start, 0), axis=1, keepdims=True)
    pos0_ref[...] = pos[:S, :]
    pos1_ref[...] = pos[S:, :]

    bcnt = padded // BT
    bend = (start // BT) + bcnt
    total = jnp.sum(bcnt, axis=1, keepdims=True)
    laste = jnp.max(jnp.where(bcnt > 0, lax.broadcasted_iota(jnp.int32, (1, E), 1), 0),
                    axis=1, keepdims=True)
    bio = lax.broadcasted_iota(jnp.int32, (NB, E), 0)
    cnt_before = jnp.sum((jnp.broadcast_to(bend, (NB, E)) <= bio).astype(jnp.int32),
                         axis=1, keepdims=True)
    bvalid = (lax.broadcasted_iota(jnp.int32, (NB, 1), 0) < total).astype(jnp.int32)
    blk_e_ref[...] = jnp.where(bvalid > 0, cnt_before, laste)
    blk_v_ref[...] = bvalid


def _gate_route(x2, emb, Wq, bq, Wk, bk, Wv, bv, Wo, bo, Wg, bg):
    return pl.pallas_call(
        _gate_route_body,
        out_shape=[
            jax.ShapeDtypeStruct((S, K), jnp.float32),
            jax.ShapeDtypeStruct((S, 1), jnp.int32),
            jax.ShapeDtypeStruct((S, 1), jnp.int32),
            jax.ShapeDtypeStruct((NB, 1), jnp.int32),
            jax.ShapeDtypeStruct((NB, 1), jnp.int32),
        ],
    )(x2, emb, Wq, bq, Wk, bk, Wv, bv, Wo, bo, Wg, bg)


TOK_PER = S // NW


@functools.cache
def _sc_kernels():
    mesh = plsc.VectorSubcoreMesh(core_axis_name="c", subcore_axis_name="s")

    @functools.partial(
        pl.kernel,
        out_type=jax.ShapeDtypeStruct((PMAX, D), jnp.float32),
        mesh=mesh,
        scratch_types=[
            pltpu.VMEM((TOK_PER,), jnp.int32),
            pltpu.VMEM((TOK_PER,), jnp.int32),
            pltpu.VMEM((TOK_PER, D), jnp.float32),
            pltpu.SemaphoreType.DMA,
        ],
    )
    def _dispatch(x_hbm, pos0_hbm, pos1_hbm, xg_hbm, pos0_v, pos1_v, rows_v,
                  sem):
        wid = lax.axis_index("s") * 2 + lax.axis_index("c")
        base = wid * TOK_PER
        pltpu.sync_copy(pos0_hbm.at[pl.ds(base, TOK_PER)], pos0_v)
        pltpu.sync_copy(pos1_hbm.at[pl.ds(base, TOK_PER)], pos1_v)
        pltpu.sync_copy(x_hbm.at[pl.ds(base, TOK_PER)], rows_v)
        d0 = pltpu.async_copy(rows_v, xg_hbm.at[pos0_v], sem)
        d1 = pltpu.async_copy(rows_v, xg_hbm.at[pos1_v], sem)
        d0.wait()
        d1.wait()

    @functools.partial(
        pl.kernel,
        out_type=(
            jax.ShapeDtypeStruct((S, D), jnp.float32),
            jax.ShapeDtypeStruct((S, D), jnp.float32),
        ),
        mesh=mesh,
        scratch_types=[
            pltpu.VMEM((TOK_PER,), jnp.int32),
            pltpu.VMEM((TOK_PER,), jnp.int32),
            pltpu.VMEM((TOK_PER, D), jnp.float32),
            pltpu.VMEM((TOK_PER, D), jnp.float32),
            pltpu.SemaphoreType.DMA,
        ],
    )
    def _combine_gather(y_hbm, pos0_hbm, pos1_hbm, y0_hbm, y1_hbm, pos0_v,
                        pos1_v, rows0_v, rows1_v, sem):
        wid = lax.axis_index("s") * 2 + lax.axis_index("c")
        base = wid * TOK_PER
        pltpu.sync_copy(pos0_hbm.at[pl.ds(base, TOK_PER)], pos0_v)
        pltpu.sync_copy(pos1_hbm.at[pl.ds(base, TOK_PER)], pos1_v)
        g0 = pltpu.async_copy(y_hbm.at[pos0_v], rows0_v, sem)
        g1 = pltpu.async_copy(y_hbm.at[pos1_v], rows1_v, sem)
        g0.wait()
        g1.wait()
        pltpu.sync_copy(rows0_v, y0_hbm.at[pl.ds(base, TOK_PER)])
        pltpu.sync_copy(rows1_v, y1_hbm.at[pl.ds(base, TOK_PER)])

    return _dispatch, _combine_gather


def _moe_body(blk_e_ref, blk_v_ref, xg_ref, w1_ref, b1_ref, w2_ref, b2_ref,
              y_ref):
    i = pl.program_id(0)

    @pl.when(blk_v_ref[i] > 0)
    def _():
        xb = xg_ref[...].astype(jnp.bfloat16)
        w1 = w1_ref[0].astype(jnp.bfloat16)
        h = jnp.dot(xb, w1, preferred_element_type=jnp.float32)
        h = jnp.maximum(h + b1_ref[0], 0.0)
        y = jnp.dot(h.astype(jnp.bfloat16), w2_ref[0].astype(jnp.bfloat16),
                    preferred_element_type=jnp.float32)
        y_ref[...] = y + b2_ref[0]


def _moe(blk_e, blk_v, xg, W1, b1, W2, b2):
    grid_spec = pltpu.PrefetchScalarGridSpec(
        num_scalar_prefetch=2,
        grid=(NB,),
        in_specs=[
            pl.BlockSpec((BT, D), lambda i, be, bv: (i, 0)),
            pl.BlockSpec((1, D, HID), lambda i, be, bv: (be[i], 0, 0)),
            pl.BlockSpec((1, 1, HID), lambda i, be, bv: (be[i], 0, 0)),
            pl.BlockSpec((1, HID, D), lambda i, be, bv: (be[i], 0, 0)),
            pl.BlockSpec((1, 1, D), lambda i, be, bv: (be[i], 0, 0)),
        ],
        out_specs=pl.BlockSpec((BT, D), lambda i, be, bv: (i, 0)),
    )
    return pl.pallas_call(
        _moe_body,
        grid_spec=grid_spec,
        out_shape=jax.ShapeDtypeStruct((PMAX, D), jnp.float32),
    )(blk_e, blk_v, xg, W1, b1, W2, b2)


def _comb_body(y0_ref, y1_ref, w_ref, o_ref):
    w = w_ref[...]
    o_ref[...] = y0_ref[...] * w[:, 0:1] + y1_ref[...] * w[:, 1:2]


def _comb(y0, y1, topw):
    return pl.pallas_call(
        _comb_body,
        grid=(S // GT,),
        in_specs=[
            pl.BlockSpec((GT, D), lambda i: (i, 0)),
            pl.BlockSpec((GT, D), lambda i: (i, 0)),
            pl.BlockSpec((GT, K), lambda i: (i, 0)),
        ],
        out_specs=pl.BlockSpec((GT, D), lambda i: (i, 0)),
        out_shape=jax.ShapeDtypeStruct((S, D), jnp.float32),
    )(y0, y1, topw)


def kernel(x, expert_emb, Wq, bq, Wk, bk, Wv, bv, Wo, bo, Wg, bg, W1, b1, W2,
           b2):
    x2 = x.reshape(S, D)
    topw, pos0c, pos1c, blk_e2, blk_v2 = _gate_route(
        x2, expert_emb, Wq, bq.reshape(1, D), Wk, bk.reshape(1, D), Wv,
        bv.reshape(1, D), Wo, bo.reshape(1, D), Wg, bg.reshape(1, E))
    pos0 = pos0c.reshape(S)
    pos1 = pos1c.reshape(S)
    dispatch_fn, combine_fn = _sc_kernels()
    xg = dispatch_fn(x2, pos0, pos1)
    y = _moe(blk_e2.reshape(NB), blk_v2.reshape(NB), xg, W1,
             b1.reshape(E, 1, HID), W2, b2.reshape(E, 1, D))
    y0, y1 = combine_fn(y, pos0, pos1)
    out = _comb(y0, y1, topw)
    return out.reshape(1, S, D)


# speedup vs baseline: 1.6592x; 1.0458x over previous
import functools

import jax
import jax.numpy as jnp
from jax import lax
from jax.experimental import pallas as pl
from jax.experimental.pallas import tpu as pltpu
from jax.experimental.pallas import tpu_sc as plsc

S, D, E, K, H, HD, HID = 2048, 768, 8, 2, 12, 64, 3072
GT = 512
BT = 256
HC = 2
HIDC = HID // HC
NPAIR = S * K
NB = NPAIR // BT + E
PMAX = NB * BT
NW = 32
PAIRS_PER = NPAIR // NW


def _gate_route_body(x_ref, emb_ref, wq_ref, bq_ref, wk_ref, bkt_ref, wv_ref,
                     bv_ref, wo_ref, bo_ref, wg_ref, bg_ref, topw_ref,
                     pos0_ref, pos1_ref, blk_e_ref, blk_v_ref):
    xb = x_ref[...]
    q = jnp.dot(xb, wq_ref[...], preferred_element_type=jnp.float32) + bq_ref[...]
    emb = emb_ref[...]
    kT = lax.dot_general(wk_ref[...], emb, (((0,), (1,)), ((), ())),
                         preferred_element_type=jnp.float32) + bkt_ref[...]
    v = jnp.dot(emb, wv_ref[...], preferred_element_type=jnp.float32) + bv_ref[...]
    wog = jnp.dot(wo_ref[...], wg_ref[...], preferred_element_type=jnp.float32)
    bog = jnp.dot(bo_ref[...], wg_ref[...], preferred_element_type=jnp.float32) + bg_ref[...]

    HE = H * E
    cols = []
    for h in range(H):
        kTh = kT[h * HD:(h + 1) * HD, :]
        blocks = []
        if h > 0:
            blocks.append(jnp.zeros((h * HD, E), jnp.float32))
        blocks.append(kTh)
        if h < H - 1:
            blocks.append(jnp.zeros((D - (h + 1) * HD, E), jnp.float32))
        cols.append(jnp.concatenate(blocks, axis=0))
    kbd = jnp.concatenate(cols, axis=1)
    rows = []
    for h in range(H):
        vh = v[:, h * HD:(h + 1) * HD]
        blocks = []
        if h > 0:
            blocks.append(jnp.zeros((E, h * HD), jnp.float32))
        blocks.append(vh)
        if h < H - 1:
            blocks.append(jnp.zeros((E, D - (h + 1) * HD), jnp.float32))
        rows.append(jnp.concatenate(blocks, axis=1))
    vbd = jnp.concatenate(rows, axis=0)
    vwog = jnp.dot(vbd, wog, preferred_element_type=jnp.float32)

    inv = 1.0 / (HD ** 0.5)
    scores = jnp.dot(q, kbd, preferred_element_type=jnp.float32) * inv
    pexp = jnp.exp(scores)
    seg = (lax.broadcasted_iota(jnp.int32, (HE, HE), 0) // E ==
           lax.broadcasted_iota(jnp.int32, (HE, HE), 1) // E).astype(jnp.float32)
    denom = jnp.dot(pexp, seg, preferred_element_type=jnp.float32)
    attn = pexp / denom
    logits = jnp.dot(attn, vwog, preferred_element_type=jnp.float32) + bog

    logits = logits - jnp.max(logits, axis=1, keepdims=True)
    ep = jnp.exp(logits)
    probs = ep / jnp.sum(ep, axis=1, keepdims=True)

    eio = lax.broadcasted_iota(jnp.int32, (S, E), 1)
    m1 = jnp.max(probs, axis=1, keepdims=True)
    i1 = jnp.min(jnp.where(probs == m1, eio, E), axis=1, keepdims=True)
    probs2 = jnp.where(eio == i1, -1.0, probs)
    m2 = jnp.max(probs2, axis=1, keepdims=True)
    i2 = jnp.min(jnp.where(probs2 == m2, eio, E), axis=1, keepdims=True)
    denom = m1 + m2 + 1e-9
    topw_ref[...] = jnp.concatenate([m1 / denom, m2 / denom], axis=1)

    oh = jnp.concatenate([(i1 == eio[:, :]).astype(jnp.int32),
                          (i2 == eio[:, :]).astype(jnp.int32)], axis=0)
    run = oh
    sh = 1
    while sh < NPAIR:
        z = jnp.zeros((sh, E), jnp.int32)
        run = run + jnp.concatenate([z, run[:NPAIR - sh, :]], axis=0)
        sh *= 2
    rank = run - oh
    counts = run[NPAIR - 1:NPAIR, :]
    padded = ((counts + (BT - 1)) // BT) * BT
    r8 = lax.broadcasted_iota(jnp.int32, (E, E), 0)
    c8 = lax.broadcasted_iota(jnp.int32, (E, E), 1)
    ltri = (r8 < c8).astype(jnp.float32)
    start = jnp.dot(padded.astype(jnp.float32), ltri,
                    preferred_element_type=jnp.float32).astype(jnp.int32)
    pos = jnp.sum(jnp.where(oh > 0, rank + start, 0), axis=1, keepdims=True)
    pos0_ref[...] = pos[:S, :]
    pos1_ref[...] = pos[S:, :]

    bcnt = padded // BT
    bend = (start // BT) + bcnt
    total = jnp.sum(bcnt, axis=1, keepdims=True)
    laste = jnp.max(jnp.where(bcnt > 0, lax.broadcasted_iota(jnp.int32, (1, E), 1), 0),
                    axis=1, keepdims=True)
    bio = lax.broadcasted_iota(jnp.int32, (NB, E), 0)
    cnt_before = jnp.sum((jnp.broadcast_to(bend, (NB, E)) <= bio).astype(jnp.int32),
                         axis=1, keepdims=True)
    bvalid = (lax.broadcasted_iota(jnp.int32, (NB, 1), 0) < total).astype(jnp.int32)
    blk_e_ref[...] = jnp.where(bvalid > 0, cnt_before, laste)
    blk_v_ref[...] = bvalid


def _gate_route(x2, emb, Wq, bq, Wk, bk, Wv, bv, Wo, bo, Wg, bg):
    return pl.pallas_call(
        _gate_route_body,
        out_shape=[
            jax.ShapeDtypeStruct((S, K), jnp.float32),
            jax.ShapeDtypeStruct((S, 1), jnp.int32),
            jax.ShapeDtypeStruct((S, 1), jnp.int32),
            jax.ShapeDtypeStruct((NB, 1), jnp.int32),
            jax.ShapeDtypeStruct((NB, 1), jnp.int32),
        ],
    )(x2, emb, Wq, bq, Wk, bk, Wv, bv, Wo, bo, Wg, bg)


TOK_PER = S // NW


@functools.cache
def _sc_kernels():
    mesh = plsc.VectorSubcoreMesh(core_axis_name="c", subcore_axis_name="s")

    @functools.partial(
        pl.kernel,
        out_type=jax.ShapeDtypeStruct((PMAX, D), jnp.float32),
        mesh=mesh,
        scratch_types=[
            pltpu.VMEM((TOK_PER,), jnp.int32),
            pltpu.VMEM((TOK_PER,), jnp.int32),
            pltpu.VMEM((TOK_PER, D), jnp.float32),
            pltpu.SemaphoreType.DMA,
        ],
    )
    def _dispatch(x_hbm, pos0_hbm, pos1_hbm, xg_hbm, pos0_v, pos1_v, rows_v,
                  sem):
        wid = lax.axis_index("s") * 2 + lax.axis_index("c")
        base = wid * TOK_PER
        pltpu.sync_copy(pos0_hbm.at[pl.ds(base, TOK_PER)], pos0_v)
        pltpu.sync_copy(pos1_hbm.at[pl.ds(base, TOK_PER)], pos1_v)
        pltpu.sync_copy(x_hbm.at[pl.ds(base, TOK_PER)], rows_v)
        d0 = pltpu.async_copy(rows_v, xg_hbm.at[pos0_v], sem)
        d1 = pltpu.async_copy(rows_v, xg_hbm.at[pos1_v], sem)
        d0.wait()
        d1.wait()

    @functools.partial(
        pl.kernel,
        out_type=(
            jax.ShapeDtypeStruct((S, D), jnp.float32),
            jax.ShapeDtypeStruct((S, D), jnp.float32),
        ),
        mesh=mesh,
        scratch_types=[
            pltpu.VMEM((TOK_PER,), jnp.int32),
            pltpu.VMEM((TOK_PER,), jnp.int32),
            pltpu.VMEM((TOK_PER, D), jnp.float32),
            pltpu.VMEM((TOK_PER, D), jnp.float32),
            pltpu.SemaphoreType.DMA,
        ],
    )
    def _combine_gather(y_hbm, pos0_hbm, pos1_hbm, y0_hbm, y1_hbm, pos0_v,
                        pos1_v, rows0_v, rows1_v, sem):
        wid = lax.axis_index("s") * 2 + lax.axis_index("c")
        base = wid * TOK_PER
        pltpu.sync_copy(pos0_hbm.at[pl.ds(base, TOK_PER)], pos0_v)
        pltpu.sync_copy(pos1_hbm.at[pl.ds(base, TOK_PER)], pos1_v)
        g0 = pltpu.async_copy(y_hbm.at[pos0_v], rows0_v, sem)
        g1 = pltpu.async_copy(y_hbm.at[pos1_v], rows1_v, sem)
        g0.wait()
        g1.wait()
        pltpu.sync_copy(rows0_v, y0_hbm.at[pl.ds(base, TOK_PER)])
        pltpu.sync_copy(rows1_v, y1_hbm.at[pl.ds(base, TOK_PER)])

    return _dispatch, _combine_gather


def _moe_body(blk_e_ref, blk_v_ref, xg_ref, w1_ref, b1_ref, w2_ref, b2_ref,
              y_ref):
    i = pl.program_id(0)

    @pl.when(blk_v_ref[i] > 0)
    def _():
        h = jnp.dot(xg_ref[...], w1_ref[0], preferred_element_type=jnp.float32)
        h = jnp.maximum(h + b1_ref[0], 0.0)
        y = jnp.dot(h, w2_ref[0], preferred_element_type=jnp.float32)
        y_ref[...] = y + b2_ref[0]


def _moe(blk_e, blk_v, xg, W1, b1, W2, b2):
    grid_spec = pltpu.PrefetchScalarGridSpec(
        num_scalar_prefetch=2,
        grid=(NB,),
        in_specs=[
            pl.BlockSpec((BT, D), lambda i, be, bv: (i, 0)),
            pl.BlockSpec((1, D, HID), lambda i, be, bv: (be[i], 0, 0)),
            pl.BlockSpec((1, 1, HID), lambda i, be, bv: (be[i], 0, 0)),
            pl.BlockSpec((1, HID, D), lambda i, be, bv: (be[i], 0, 0)),
            pl.BlockSpec((1, 1, D), lambda i, be, bv: (be[i], 0, 0)),
        ],
        out_specs=pl.BlockSpec((BT, D), lambda i, be, bv: (i, 0)),
    )
    return pl.pallas_call(
        _moe_body,
        grid_spec=grid_spec,
        out_shape=jax.ShapeDtypeStruct((PMAX, D), jnp.float32),
    )(blk_e, blk_v, xg, W1, b1, W2, b2)


def _comb_body(y0_ref, y1_ref, w_ref, o_ref):
    w = w_ref[...]
    o_ref[...] = y0_ref[...] * w[:, 0:1] + y1_ref[...] * w[:, 1:2]


def _comb(y0, y1, topw):
    return pl.pallas_call(
        _comb_body,
        grid=(S // GT,),
        in_specs=[
            pl.BlockSpec((GT, D), lambda i: (i, 0)),
            pl.BlockSpec((GT, D), lambda i: (i, 0)),
            pl.BlockSpec((GT, K), lambda i: (i, 0)),
        ],
        out_specs=pl.BlockSpec((GT, D), lambda i: (i, 0)),
        out_shape=jax.ShapeDtypeStruct((S, D), jnp.float32),
    )(y0, y1, topw)


def kernel(x, expert_emb, Wq, bq, Wk, bk, Wv, bv, Wo, bo, Wg, bg, W1, b1, W2,
           b2):
    x2 = x.reshape(S, D)
    topw, pos0c, pos1c, blk_e2, blk_v2 = _gate_route(
        x2, expert_emb, Wq, bq.reshape(1, D), Wk, bk.reshape(D, 1), Wv,
        bv.reshape(1, D), Wo, bo.reshape(1, D), Wg, bg.reshape(1, E))
    pos0 = pos0c.reshape(S)
    pos1 = pos1c.reshape(S)
    dispatch_fn, combine_fn = _sc_kernels()
    xg = dispatch_fn(x2, pos0, pos1)
    y = _moe(blk_e2.reshape(NB), blk_v2.reshape(NB), xg, W1,
             b1.reshape(E, 1, HID), W2, b2.reshape(E, 1, D))
    y0, y1 = combine_fn(y, pos0, pos1)
    out = _comb(y0, y1, topw)
    return out.reshape(1, S, D)


# speedup vs baseline: 1.6732x; 1.0084x over previous
import functools

import jax
import jax.numpy as jnp
from jax import lax
from jax.experimental import pallas as pl
from jax.experimental.pallas import tpu as pltpu
from jax.experimental.pallas import tpu_sc as plsc

S, D, E, K, H, HD, HID = 2048, 768, 8, 2, 12, 64, 3072
GT = 512
BT = 256
HC = 2
HIDC = HID // HC
NPAIR = S * K
NB = NPAIR // BT + E
PMAX = NB * BT
NW = 32
PAIRS_PER = NPAIR // NW


def _gate_route_body(x_ref, emb_ref, wq_ref, bq_ref, wk_ref, bkt_ref, wv_ref,
                     bv_ref, wo_ref, bo_ref, wg_ref, bg_ref, topw_ref,
                     pos0_ref, pos1_ref, blk_e_ref, blk_v_ref):
    xb = x_ref[...]
    q = jnp.dot(xb, wq_ref[...], preferred_element_type=jnp.float32) + bq_ref[...]
    emb = emb_ref[...]
    kT = lax.dot_general(wk_ref[...], emb, (((0,), (1,)), ((), ())),
                         preferred_element_type=jnp.float32) + bkt_ref[...]
    v = jnp.dot(emb, wv_ref[...], preferred_element_type=jnp.float32) + bv_ref[...]
    wog = jnp.dot(wo_ref[...], wg_ref[...], preferred_element_type=jnp.float32)
    bog = jnp.dot(bo_ref[...], wg_ref[...], preferred_element_type=jnp.float32) + bg_ref[...]

    HE = H * E
    cols = []
    for h in range(H):
        kTh = kT[h * HD:(h + 1) * HD, :]
        blocks = []
        if h > 0:
            blocks.append(jnp.zeros((h * HD, E), jnp.float32))
        blocks.append(kTh)
        if h < H - 1:
            blocks.append(jnp.zeros((D - (h + 1) * HD, E), jnp.float32))
        cols.append(jnp.concatenate(blocks, axis=0))
    kbd = jnp.concatenate(cols, axis=1)
    rows = []
    for h in range(H):
        vh = v[:, h * HD:(h + 1) * HD]
        blocks = []
        if h > 0:
            blocks.append(jnp.zeros((E, h * HD), jnp.float32))
        blocks.append(vh)
        if h < H - 1:
            blocks.append(jnp.zeros((E, D - (h + 1) * HD), jnp.float32))
        rows.append(jnp.concatenate(blocks, axis=1))
    vbd = jnp.concatenate(rows, axis=0)
    vwog = jnp.dot(vbd, wog, preferred_element_type=jnp.float32)

    inv = 1.0 / (HD ** 0.5)
    scores = jnp.dot(q, kbd, preferred_element_type=jnp.float32) * inv
    pexp = jnp.exp(scores)
    seg = (lax.broadcasted_iota(jnp.int32, (HE, HE), 0) // E ==
           lax.broadcasted_iota(jnp.int32, (HE, HE), 1) // E).astype(jnp.float32)
    denom = jnp.dot(pexp, seg, preferred_element_type=jnp.float32)
    attn = pexp / denom
    logits = jnp.dot(attn, vwog, preferred_element_type=jnp.float32) + bog

    logits = logits - jnp.max(logits, axis=1, keepdims=True)
    ep = jnp.exp(logits)
    probs = ep / jnp.sum(ep, axis=1, keepdims=True)

    eio = lax.broadcasted_iota(jnp.int32, (S, E), 1)
    m1 = jnp.max(probs, axis=1, keepdims=True)
    i1 = jnp.min(jnp.where(probs == m1, eio, E), axis=1, keepdims=True)
    probs2 = jnp.where(eio == i1, -1.0, probs)
    m2 = jnp.max(probs2, axis=1, keepdims=True)
    i2 = jnp.min(jnp.where(probs2 == m2, eio, E), axis=1, keepdims=True)
    denom = m1 + m2 + 1e-9
    topw_ref[...] = jnp.concatenate([m1 / denom, m2 / denom], axis=1)

    oh = jnp.concatenate([(i1 == eio[:, :]).astype(jnp.int32),
                          (i2 == eio[:, :]).astype(jnp.int32)], axis=0)
    run = oh
    sh = 1
    while sh < NPAIR:
        z = jnp.zeros((sh, E), jnp.int32)
        run = run + jnp.concatenate([z, run[:NPAIR - sh, :]], axis=0)
        sh *= 2
    rank = run - oh
    counts = run[NPAIR - 1:NPAIR, :]
    padded = ((counts + (BT - 1)) // BT) * BT
    r8 = lax.broadcasted_iota(jnp.int32, (E, E), 0)
    c8 = lax.broadcasted_iota(jnp.int32, (E, E), 1)
    ltri = (r8 < c8).astype(jnp.float32)
    start = jnp.dot(padded.astype(jnp.float32), ltri,
                    preferred_element_type=jnp.float32).astype(jnp.int32)
    pos = jnp.sum(jnp.where(oh > 0, rank + start, 0), axis=1, keepdims=True)
    pos0_ref[...] = pos[:S, :]
    pos1_ref[...] = pos[S:, :]

    bcnt = padded // BT
    bend = (start // BT) + bcnt
    total = jnp.sum(bcnt, axis=1, keepdims=True)
    laste = jnp.max(jnp.where(bcnt > 0, lax.broadcasted_iota(jnp.int32, (1, E), 1), 0),
                    axis=1, keepdims=True)
    bio = lax.broadcasted_iota(jnp.int32, (NB, E), 0)
    cnt_before = jnp.sum((jnp.broadcast_to(bend, (NB, E)) <= bio).astype(jnp.int32),
                         axis=1, keepdims=True)
    bvalid = (lax.broadcasted_iota(jnp.int32, (NB, 1), 0) < total).astype(jnp.int32)
    blk_e_ref[...] = jnp.where(bvalid > 0, cnt_before, laste)
    blk_v_ref[...] = bvalid


def _gate_route(x2, emb, Wq, bq, Wk, bk, Wv, bv, Wo, bo, Wg, bg):
    return pl.pallas_call(
        _gate_route_body,
        out_shape=[
            jax.ShapeDtypeStruct((S, K), jnp.float32),
            jax.ShapeDtypeStruct((S, 1), jnp.int32),
            jax.ShapeDtypeStruct((S, 1), jnp.int32),
            jax.ShapeDtypeStruct((NB, 1), jnp.int32),
            jax.ShapeDtypeStruct((NB, 1), jnp.int32),
        ],
    )(x2, emb, Wq, bq, Wk, bk, Wv, bv, Wo, bo, Wg, bg)


TOK_PER = S // NW


@functools.cache
def _sc_kernels():
    mesh = plsc.VectorSubcoreMesh(core_axis_name="c", subcore_axis_name="s")

    @functools.partial(
        pl.kernel,
        out_type=jax.ShapeDtypeStruct((PMAX, D), jnp.float32),
        mesh=mesh,
        scratch_types=[
            pltpu.VMEM((TOK_PER,), jnp.int32),
            pltpu.VMEM((TOK_PER,), jnp.int32),
            pltpu.VMEM((TOK_PER, D), jnp.float32),
            pltpu.SemaphoreType.DMA,
            pltpu.SemaphoreType.DMA,
            pltpu.SemaphoreType.DMA,
            pltpu.SemaphoreType.DMA,
        ],
    )
    def _dispatch(x_hbm, pos0_hbm, pos1_hbm, xg_hbm, pos0_v, pos1_v, rows_v,
                  sem_p0, sem_p1, sem_x, sem):
        wid = lax.axis_index("s") * 2 + lax.axis_index("c")
        base = wid * TOK_PER
        p0 = pltpu.async_copy(pos0_hbm.at[pl.ds(base, TOK_PER)], pos0_v, sem_p0)
        p1 = pltpu.async_copy(pos1_hbm.at[pl.ds(base, TOK_PER)], pos1_v, sem_p1)
        dx = pltpu.async_copy(x_hbm.at[pl.ds(base, TOK_PER)], rows_v, sem_x)
        p0.wait()
        dx.wait()
        d0 = pltpu.async_copy(rows_v, xg_hbm.at[pos0_v], sem)
        p1.wait()
        d1 = pltpu.async_copy(rows_v, xg_hbm.at[pos1_v], sem)
        d0.wait()
        d1.wait()

    @functools.partial(
        pl.kernel,
        out_type=(
            jax.ShapeDtypeStruct((S, D), jnp.float32),
            jax.ShapeDtypeStruct((S, D), jnp.float32),
        ),
        mesh=mesh,
        scratch_types=[
            pltpu.VMEM((TOK_PER,), jnp.int32),
            pltpu.VMEM((TOK_PER,), jnp.int32),
            pltpu.VMEM((TOK_PER, D), jnp.float32),
            pltpu.VMEM((TOK_PER, D), jnp.float32),
            pltpu.SemaphoreType.DMA,
            pltpu.SemaphoreType.DMA,
            pltpu.SemaphoreType.DMA,
            pltpu.SemaphoreType.DMA,
        ],
    )
    def _combine_gather(y_hbm, pos0_hbm, pos1_hbm, y0_hbm, y1_hbm, pos0_v,
                        pos1_v, rows0_v, rows1_v, sem_p0, sem_p1, sem0, sem1):
        wid = lax.axis_index("s") * 2 + lax.axis_index("c")
        base = wid * TOK_PER
        p0 = pltpu.async_copy(pos0_hbm.at[pl.ds(base, TOK_PER)], pos0_v, sem_p0)
        p1 = pltpu.async_copy(pos1_hbm.at[pl.ds(base, TOK_PER)], pos1_v, sem_p1)
        p0.wait()
        g0 = pltpu.async_copy(y_hbm.at[pos0_v], rows0_v, sem0)
        p1.wait()
        g1 = pltpu.async_copy(y_hbm.at[pos1_v], rows1_v, sem1)
        g0.wait()
        w0 = pltpu.async_copy(rows0_v, y0_hbm.at[pl.ds(base, TOK_PER)], sem0)
        g1.wait()
        w1 = pltpu.async_copy(rows1_v, y1_hbm.at[pl.ds(base, TOK_PER)], sem1)
        w0.wait()
        w1.wait()

    return _dispatch, _combine_gather


def _moe_body(blk_e_ref, blk_v_ref, xg_ref, w1_ref, b1_ref, w2_ref, b2_ref,
              y_ref):
    i = pl.program_id(0)

    @pl.when(blk_v_ref[i] > 0)
    def _():
        h = jnp.dot(xg_ref[...], w1_ref[0], preferred_element_type=jnp.float32)
        h = jnp.maximum(h + b1_ref[0], 0.0)
        y = jnp.dot(h, w2_ref[0], preferred_element_type=jnp.float32)
        y_ref[...] = y + b2_ref[0]


def _moe(blk_e, blk_v, xg, W1, b1, W2, b2):
    grid_spec = pltpu.PrefetchScalarGridSpec(
        num_scalar_prefetch=2,
        grid=(NB,),
        in_specs=[
            pl.BlockSpec((BT, D), lambda i, be, bv: (i, 0)),
            pl.BlockSpec((1, D, HID), lambda i, be, bv: (be[i], 0, 0)),
            pl.BlockSpec((1, 1, HID), lambda i, be, bv: (be[i], 0, 0)),
            pl.BlockSpec((1, HID, D), lambda i, be, bv: (be[i], 0, 0)),
            pl.BlockSpec((1, 1, D), lambda i, be, bv: (be[i], 0, 0)),
        ],
        out_specs=pl.BlockSpec((BT, D), lambda i, be, bv: (i, 0)),
    )
    return pl.pallas_call(
        _moe_body,
        grid_spec=grid_spec,
        out_shape=jax.ShapeDtypeStruct((PMAX, D), jnp.float32),
    )(blk_e, blk_v, xg, W1, b1, W2, b2)


def _comb_body(y0_ref, y1_ref, w_ref, o_ref):
    w = w_ref[...]
    o_ref[...] = y0_ref[...] * w[:, 0:1] + y1_ref[...] * w[:, 1:2]


def _comb(y0, y1, topw):
    return pl.pallas_call(
        _comb_body,
        grid=(S // GT,),
        in_specs=[
            pl.BlockSpec((GT, D), lambda i: (i, 0)),
            pl.BlockSpec((GT, D), lambda i: (i, 0)),
            pl.BlockSpec((GT, K), lambda i: (i, 0)),
        ],
        out_specs=pl.BlockSpec((GT, D), lambda i: (i, 0)),
        out_shape=jax.ShapeDtypeStruct((S, D), jnp.float32),
    )(y0, y1, topw)


def kernel(x, expert_emb, Wq, bq, Wk, bk, Wv, bv, Wo, bo, Wg, bg, W1, b1, W2,
           b2):
    x2 = x.reshape(S, D)
    topw, pos0c, pos1c, blk_e2, blk_v2 = _gate_route(
        x2, expert_emb, Wq, bq.reshape(1, D), Wk, bk.reshape(D, 1), Wv,
        bv.reshape(1, D), Wo, bo.reshape(1, D), Wg, bg.reshape(1, E))
    pos0 = pos0c.reshape(S)
    pos1 = pos1c.reshape(S)
    dispatch_fn, combine_fn = _sc_kernels()
    xg = dispatch_fn(x2, pos0, pos1)
    y = _moe(blk_e2.reshape(NB), blk_v2.reshape(NB), xg, W1,
             b1.reshape(E, 1, HID), W2, b2.reshape(E, 1, D))
    y0, y1 = combine_fn(y, pos0, pos1)
    out = _comb(y0, y1, topw)
    return out.reshape(1, S, D)


# speedup vs baseline: 1.7303x; 1.0341x over previous
import functools

import jax
import jax.numpy as jnp
from jax import lax
from jax.experimental import pallas as pl
from jax.experimental.pallas import tpu as pltpu
from jax.experimental.pallas import tpu_sc as plsc

S, D, E, K, H, HD, HID = 2048, 768, 8, 2, 12, 64, 3072
GT = 512
BT = 256
HC = 2
HIDC = HID // HC
NPAIR = S * K
NB = NPAIR // BT + E
PMAX = NB * BT
NW = 32
PAIRS_PER = NPAIR // NW


def _gate_route_body(x_ref, emb_ref, wq_ref, bq_ref, wk_ref, bkt_ref, wv_ref,
                     bv_ref, wo_ref, bo_ref, wg_ref, bg_ref, w0r_ref, w1r_ref,
                     pos0_ref, pos1_ref, blk_e_ref, blk_v_ref):
    xb = x_ref[...]
    q = jnp.dot(xb, wq_ref[...], preferred_element_type=jnp.float32) + bq_ref[...]
    emb = emb_ref[...]
    kT = lax.dot_general(wk_ref[...], emb, (((0,), (1,)), ((), ())),
                         preferred_element_type=jnp.float32) + bkt_ref[...]
    v = jnp.dot(emb, wv_ref[...], preferred_element_type=jnp.float32) + bv_ref[...]
    wog = jnp.dot(wo_ref[...], wg_ref[...], preferred_element_type=jnp.float32)
    bog = jnp.dot(bo_ref[...], wg_ref[...], preferred_element_type=jnp.float32) + bg_ref[...]

    HE = H * E
    cols = []
    for h in range(H):
        kTh = kT[h * HD:(h + 1) * HD, :]
        blocks = []
        if h > 0:
            blocks.append(jnp.zeros((h * HD, E), jnp.float32))
        blocks.append(kTh)
        if h < H - 1:
            blocks.append(jnp.zeros((D - (h + 1) * HD, E), jnp.float32))
        cols.append(jnp.concatenate(blocks, axis=0))
    kbd = jnp.concatenate(cols, axis=1)
    rows = []
    for h in range(H):
        vh = v[:, h * HD:(h + 1) * HD]
        blocks = []
        if h > 0:
            blocks.append(jnp.zeros((E, h * HD), jnp.float32))
        blocks.append(vh)
        if h < H - 1:
            blocks.append(jnp.zeros((E, D - (h + 1) * HD), jnp.float32))
        rows.append(jnp.concatenate(blocks, axis=1))
    vbd = jnp.concatenate(rows, axis=0)
    vwog = jnp.dot(vbd, wog, preferred_element_type=jnp.float32)

    inv = 1.0 / (HD ** 0.5)
    scores = jnp.dot(q, kbd, preferred_element_type=jnp.float32) * inv
    pexp = jnp.exp(scores)
    seg = (lax.broadcasted_iota(jnp.int32, (HE, HE), 0) // E ==
           lax.broadcasted_iota(jnp.int32, (HE, HE), 1) // E).astype(jnp.float32)
    denom = jnp.dot(pexp, seg, preferred_element_type=jnp.float32)
    attn = pexp / denom
    logits = jnp.dot(attn, vwog, preferred_element_type=jnp.float32) + bog

    logits = logits - jnp.max(logits, axis=1, keepdims=True)
    ep = jnp.exp(logits)
    probs = ep / jnp.sum(ep, axis=1, keepdims=True)

    eio = lax.broadcasted_iota(jnp.int32, (S, E), 1)
    m1 = jnp.max(probs, axis=1, keepdims=True)
    i1 = jnp.min(jnp.where(probs == m1, eio, E), axis=1, keepdims=True)
    probs2 = jnp.where(eio == i1, -1.0, probs)
    m2 = jnp.max(probs2, axis=1, keepdims=True)
    i2 = jnp.min(jnp.where(probs2 == m2, eio, E), axis=1, keepdims=True)
    denom = m1 + m2 + 1e-9
    w0r_ref[...] = jnp.broadcast_to(m1 / denom, (S, 128))
    w1r_ref[...] = jnp.broadcast_to(m2 / denom, (S, 128))

    oh = jnp.concatenate([(i1 == eio[:, :]).astype(jnp.int32),
                          (i2 == eio[:, :]).astype(jnp.int32)], axis=0)
    run = oh
    sh = 1
    while sh < NPAIR:
        z = jnp.zeros((sh, E), jnp.int32)
        run = run + jnp.concatenate([z, run[:NPAIR - sh, :]], axis=0)
        sh *= 2
    rank = run - oh
    counts = run[NPAIR - 1:NPAIR, :]
    padded = ((counts + (BT - 1)) // BT) * BT
    r8 = lax.broadcasted_iota(jnp.int32, (E, E), 0)
    c8 = lax.broadcasted_iota(jnp.int32, (E, E), 1)
    ltri = (r8 < c8).astype(jnp.float32)
    start = jnp.dot(padded.astype(jnp.float32), ltri,
                    preferred_element_type=jnp.float32).astype(jnp.int32)
    pos = jnp.sum(jnp.where(oh > 0, rank + start, 0), axis=1, keepdims=True)
    pos0_ref[...] = pos[:S, :]
    pos1_ref[...] = pos[S:, :]

    bcnt = padded // BT
    bend = (start // BT) + bcnt
    total = jnp.sum(bcnt, axis=1, keepdims=True)
    laste = jnp.max(jnp.where(bcnt > 0, lax.broadcasted_iota(jnp.int32, (1, E), 1), 0),
                    axis=1, keepdims=True)
    bio = lax.broadcasted_iota(jnp.int32, (NB, E), 0)
    cnt_before = jnp.sum((jnp.broadcast_to(bend, (NB, E)) <= bio).astype(jnp.int32),
                         axis=1, keepdims=True)
    bvalid = (lax.broadcasted_iota(jnp.int32, (NB, 1), 0) < total).astype(jnp.int32)
    blk_e_ref[...] = jnp.where(bvalid > 0, cnt_before, laste)
    blk_v_ref[...] = bvalid


def _gate_route(x2, emb, Wq, bq, Wk, bk, Wv, bv, Wo, bo, Wg, bg):
    return pl.pallas_call(
        _gate_route_body,
        out_shape=[
            jax.ShapeDtypeStruct((S, 128), jnp.float32),
            jax.ShapeDtypeStruct((S, 128), jnp.float32),
            jax.ShapeDtypeStruct((S, 1), jnp.int32),
            jax.ShapeDtypeStruct((S, 1), jnp.int32),
            jax.ShapeDtypeStruct((NB, 1), jnp.int32),
            jax.ShapeDtypeStruct((NB, 1), jnp.int32),
        ],
    )(x2, emb, Wq, bq, Wk, bk, Wv, bv, Wo, bo, Wg, bg)


TOK_PER = S // NW


@functools.cache
def _sc_kernels():
    mesh = plsc.VectorSubcoreMesh(core_axis_name="c", subcore_axis_name="s")

    @functools.partial(
        pl.kernel,
        out_type=(
            jax.ShapeDtypeStruct((PMAX, D), jnp.float32),
            jax.ShapeDtypeStruct((PMAX, 128), jnp.float32),
        ),
        mesh=mesh,
        scratch_types=[
            pltpu.VMEM((TOK_PER,), jnp.int32),
            pltpu.VMEM((TOK_PER,), jnp.int32),
            pltpu.VMEM((TOK_PER, D), jnp.float32),
            pltpu.VMEM((TOK_PER, 128), jnp.float32),
            pltpu.VMEM((TOK_PER, 128), jnp.float32),
            pltpu.SemaphoreType.DMA,
            pltpu.SemaphoreType.DMA,
            pltpu.SemaphoreType.DMA,
            pltpu.SemaphoreType.DMA,
            pltpu.SemaphoreType.DMA,
        ],
    )
    def _dispatch(x_hbm, pos0_hbm, pos1_hbm, w0r_hbm, w1r_hbm, xg_hbm,
                  wpos_hbm, pos0_v, pos1_v, rows_v, w0_v, w1_v, sem_p0,
                  sem_p1, sem_x, sem_w, sem):
        wid = lax.axis_index("s") * 2 + lax.axis_index("c")
        base = wid * TOK_PER
        p0 = pltpu.async_copy(pos0_hbm.at[pl.ds(base, TOK_PER)], pos0_v, sem_p0)
        p1 = pltpu.async_copy(pos1_hbm.at[pl.ds(base, TOK_PER)], pos1_v, sem_p1)
        dx = pltpu.async_copy(x_hbm.at[pl.ds(base, TOK_PER)], rows_v, sem_x)
        dw0 = pltpu.async_copy(w0r_hbm.at[pl.ds(base, TOK_PER)], w0_v, sem_w)
        dw1 = pltpu.async_copy(w1r_hbm.at[pl.ds(base, TOK_PER)], w1_v, sem_w)
        p0.wait()
        dx.wait()
        d0 = pltpu.async_copy(rows_v, xg_hbm.at[pos0_v], sem)
        p1.wait()
        d1 = pltpu.async_copy(rows_v, xg_hbm.at[pos1_v], sem)
        dw0.wait()
        dw1.wait()
        s0 = pltpu.async_copy(w0_v, wpos_hbm.at[pos0_v], sem_w)
        s1 = pltpu.async_copy(w1_v, wpos_hbm.at[pos1_v], sem_w)
        d0.wait()
        d1.wait()
        s0.wait()
        s1.wait()

    @functools.partial(
        pl.kernel,
        out_type=jax.ShapeDtypeStruct((S, D), jnp.float32),
        mesh=mesh,
        scratch_types=[
            pltpu.VMEM((TOK_PER,), jnp.int32),
            pltpu.VMEM((TOK_PER,), jnp.int32),
            pltpu.VMEM((TOK_PER, D), jnp.float32),
            pltpu.VMEM((TOK_PER, D), jnp.float32),
            pltpu.SemaphoreType.DMA,
            pltpu.SemaphoreType.DMA,
            pltpu.SemaphoreType.DMA,
            pltpu.SemaphoreType.DMA,
        ],
    )
    def _combine_gather(z_hbm, pos0_hbm, pos1_hbm, out_hbm, pos0_v, pos1_v,
                        rows0_v, rows1_v, sem_p0, sem_p1, sem0, sem1):
        wid = lax.axis_index("s") * 2 + lax.axis_index("c")
        base = wid * TOK_PER
        p0 = pltpu.async_copy(pos0_hbm.at[pl.ds(base, TOK_PER)], pos0_v, sem_p0)
        p1 = pltpu.async_copy(pos1_hbm.at[pl.ds(base, TOK_PER)], pos1_v, sem_p1)
        p0.wait()
        g0 = pltpu.async_copy(z_hbm.at[pos0_v], rows0_v, sem0)
        p1.wait()
        g1 = pltpu.async_copy(z_hbm.at[pos1_v], rows1_v, sem1)
        g0.wait()
        g1.wait()

        def add_row(r, _):
            for j in range(D // 16):
                sl = pl.ds(j * 16, 16)
                rows0_v[r, sl] = rows0_v[r, sl] + rows1_v[r, sl]
            return 0

        lax.fori_loop(0, TOK_PER, add_row, 0)
        pltpu.sync_copy(rows0_v, out_hbm.at[pl.ds(base, TOK_PER)])

    return _dispatch, _combine_gather


def _moe_body(blk_e_ref, blk_v_ref, xg_ref, wpos_ref, w1_ref, b1_ref, w2_ref,
              b2_ref, y_ref):
    i = pl.program_id(0)

    @pl.when(blk_v_ref[i] > 0)
    def _():
        h = jnp.dot(xg_ref[...], w1_ref[0], preferred_element_type=jnp.float32)
        h = jnp.maximum(h + b1_ref[0], 0.0)
        y = jnp.dot(h, w2_ref[0], preferred_element_type=jnp.float32)
        y_ref[...] = (y + b2_ref[0]) * wpos_ref[...][:, 0:1]


def _moe(blk_e, blk_v, xg, wpos, W1, b1, W2, b2):
    grid_spec = pltpu.PrefetchScalarGridSpec(
        num_scalar_prefetch=2,
        grid=(NB,),
        in_specs=[
            pl.BlockSpec((BT, D), lambda i, be, bv: (i, 0)),
            pl.BlockSpec((BT, 128), lambda i, be, bv: (i, 0)),
            pl.BlockSpec((1, D, HID), lambda i, be, bv: (be[i], 0, 0)),
            pl.BlockSpec((1, 1, HID), lambda i, be, bv: (be[i], 0, 0)),
            pl.BlockSpec((1, HID, D), lambda i, be, bv: (be[i], 0, 0)),
            pl.BlockSpec((1, 1, D), lambda i, be, bv: (be[i], 0, 0)),
        ],
        out_specs=pl.BlockSpec((BT, D), lambda i, be, bv: (i, 0)),
    )
    return pl.pallas_call(
        _moe_body,
        grid_spec=grid_spec,
        out_shape=jax.ShapeDtypeStruct((PMAX, D), jnp.float32),
    )(blk_e, blk_v, xg, wpos, W1, b1, W2, b2)


def kernel(x, expert_emb, Wq, bq, Wk, bk, Wv, bv, Wo, bo, Wg, bg, W1, b1, W2,
           b2):
    x2 = x.reshape(S, D)
    w0r, w1r, pos0c, pos1c, blk_e2, blk_v2 = _gate_route(
        x2, expert_emb, Wq, bq.reshape(1, D), Wk, bk.reshape(D, 1), Wv,
        bv.reshape(1, D), Wo, bo.reshape(1, D), Wg, bg.reshape(1, E))
    pos0 = pos0c.reshape(S)
    pos1 = pos1c.reshape(S)
    dispatch_fn, combine_fn = _sc_kernels()
    xg, wpos = dispatch_fn(x2, pos0, pos1, w0r, w1r)
    z = _moe(blk_e2.reshape(NB), blk_v2.reshape(NB), xg, wpos, W1,
             b1.reshape(E, 1, HID), W2, b2.reshape(E, 1, D))
    out = combine_fn(z, pos0, pos1)
    return out.reshape(1, S, D)
